# baseline (speedup 1.0000x reference)
import jax
import jax.numpy as jnp
from jax import lax
from jax.experimental import pallas as pl
from jax.experimental.pallas import tpu as pltpu
from jax.experimental.pallas import tpu_sc as plsc

B = 16384
F = 32
L = 16

_info = plsc.get_sparse_core_info()
NC, NS = _info.num_cores, _info.num_subcores
NW = NC * NS
B_PER_W = B // NW
NG = B_PER_W // L
CHUNK = 128
NCB = B_PER_W // CHUNK
NDMA = F * NCB


def _gmf_kernel(eu_hbm, ei_hbm, user_hbm, item_hbm, w_hbm, b_hbm, out_hbm,
                uidx_v, iidx_v, eu_v, ei_v, out_v, w_v, b_v, sem):
    wid = lax.axis_index("s") * NC + lax.axis_index("c")
    base = wid * B_PER_W

    pltpu.sync_copy(w_hbm, w_v)
    pltpu.sync_copy(b_hbm, b_v)
    for c in range(NCB):
        pltpu.sync_copy(user_hbm.at[pl.ds(base + c * CHUNK, CHUNK)],
                        uidx_v.at[c])
        pltpu.sync_copy(item_hbm.at[pl.ds(base + c * CHUNK, CHUNK)],
                        iidx_v.at[c])

    def fire(k, _):
        f = k // NCB
        c = k % NCB
        pltpu.async_copy(eu_hbm.at[f].at[uidx_v.at[c]], eu_v.at[k], sem)
        pltpu.async_copy(ei_hbm.at[f].at[iidx_v.at[c]], ei_v.at[k], sem)
        return 0

    lax.fori_loop(0, NDMA, fire, 0)

    def drain(k, _):
        f = k // NCB
        c = k % NCB
        pltpu.make_async_copy(
            eu_hbm.at[f].at[uidx_v.at[c]], eu_v.at[k], sem
        ).wait()
        pltpu.make_async_copy(
            ei_hbm.at[f].at[iidx_v.at[c]], ei_v.at[k], sem
        ).wait()
        return 0

    lax.fori_loop(0, NDMA, drain, 0)

    w_lo = w_v[0, pl.ds(0, L)]
    w_hi = w_v[0, pl.ds(L, L)]
    w_s = [w_lo[f] for f in range(L)] + [w_hi[f] for f in range(L)]
    bias_v = b_v[...]

    def compute(g, _):
        row0 = g // 8
        col = (g % 8) * L
        acc = bias_v
        for f in range(F):
            eu_c = eu_v[f * NCB + row0, pl.ds(col, L)]
            ei_c = ei_v[f * NCB + row0, pl.ds(col, L)]
            acc = acc + (eu_c * ei_c) * w_s[f]
        out_v[pl.ds(g * L, L)] = acc
        return 0

    lax.fori_loop(0, NG, compute, 0)

    pltpu.sync_copy(out_v, out_hbm.at[pl.ds(base, B_PER_W)])


def kernel(user, item, embed_user, embed_item, W, b):
    mesh = plsc.VectorSubcoreMesh(core_axis_name="c", subcore_axis_name="s")
    run = pl.kernel(
        _gmf_kernel,
        mesh=mesh,
        compiler_params=pltpu.CompilerParams(
            needs_layout_passes=False, use_tc_tiling_on_sc=False
        ),
        out_type=jax.ShapeDtypeStruct((B,), jnp.float32),
        scratch_types=[
            pltpu.VMEM((NCB, CHUNK), jnp.int32),
            pltpu.VMEM((NCB, CHUNK), jnp.int32),
            pltpu.VMEM((NDMA, CHUNK), jnp.float32),
            pltpu.VMEM((NDMA, CHUNK), jnp.float32),
            pltpu.VMEM((B_PER_W,), jnp.float32),
            pltpu.VMEM((1, F), jnp.float32),
            pltpu.VMEM((L,), jnp.float32),
            pltpu.SemaphoreType.DMA,
        ],
    )
    b16 = jnp.broadcast_to(b.astype(jnp.float32), (L,))
    return run(embed_user.T, embed_item.T, user, item, W, b16)


# speedup vs baseline: 18.8787x; 18.8787x over previous
import jax
import jax.numpy as jnp
from jax import lax
from jax.experimental import pallas as pl
from jax.experimental.pallas import tpu as pltpu
from jax.experimental.pallas import tpu_sc as plsc

B = 16384
F = 32
L = 16

_info = plsc.get_sparse_core_info()
NC, NS = _info.num_cores, _info.num_subcores
NW = NC * NS
B_PER_W = B // NW
WAVE = 8
NG16 = B_PER_W // L
IDX_PAD = B_PER_W + L


def _gmf_kernel(eu_hbm, ei_hbm, user_hbm, item_hbm, w_hbm, b_hbm, out_hbm,
                uidx_v, iidx_v, fidx_v, eu_sb, ei_sb, out_v, w_v, b_v, sem):
    wid = lax.axis_index("s") * NC + lax.axis_index("c")
    base = wid * B_PER_W

    pltpu.sync_copy(w_hbm, w_v)
    pltpu.sync_copy(b_hbm, b_v)
    pltpu.sync_copy(user_hbm.at[pl.ds(base, B_PER_W)],
                    uidx_v.at[pl.ds(0, B_PER_W)])
    pltpu.sync_copy(item_hbm.at[pl.ds(base, B_PER_W)],
                    iidx_v.at[pl.ds(0, B_PER_W)])

    lane = lax.iota(jnp.int32, L)
    fidx_v[pl.ds(0, L)] = lane
    fidx_v[pl.ds(L, L)] = lane + L

    w_lo = w_v[0, pl.ds(0, L)]
    w_hi = w_v[0, pl.ds(L, L)]
    bias_v = b_v[...]
    zeros = jnp.zeros((L,), jnp.int32)
    f_lo = lane
    f_hi = lane + L

    def group16(g16, _):
        acc = bias_v
        for sub in range(2):
            g = g16 * 2 + sub
            iu = uidx_v[pl.ds(g * WAVE, L)]
            ii = iidx_v[pl.ds(g * WAVE, L)]
            qu = (iu >> 7) * 128
            cu = iu & 127
            qi = (ii >> 7) * 128
            ci = ii & 127
            for jj in range(WAVE):
                pltpu.async_copy(
                    eu_hbm.at[fidx_v, pl.ds(pl.multiple_of(qu[jj], 128), 128)],
                    eu_sb.at[jj], sem,
                )
                pltpu.async_copy(
                    ei_hbm.at[fidx_v, pl.ds(pl.multiple_of(qi[jj], 128), 128)],
                    ei_sb.at[jj], sem,
                )
            for jj in range(WAVE):
                pltpu.make_async_copy(
                    eu_hbm.at[fidx_v, pl.ds(pl.multiple_of(qu[jj], 128), 128)],
                    eu_sb.at[jj], sem,
                ).wait()
                pltpu.make_async_copy(
                    ei_hbm.at[fidx_v, pl.ds(pl.multiple_of(qi[jj], 128), 128)],
                    ei_sb.at[jj], sem,
                ).wait()
            for jj in range(WAVE):
                jj_vec = jnp.full((L,), jj, jnp.int32)
                cu_vec = zeros + cu[jj]
                ci_vec = zeros + ci[jj]
                eu_l = plsc.load_gather(eu_sb, [jj_vec, f_lo, cu_vec])
                eu_h = plsc.load_gather(eu_sb, [jj_vec, f_hi, cu_vec])
                ei_l = plsc.load_gather(ei_sb, [jj_vec, f_lo, ci_vec])
                ei_h = plsc.load_gather(ei_sb, [jj_vec, f_hi, ci_vec])
                t = (eu_l * ei_l) * w_lo + (eu_h * ei_h) * w_hi
                acc = jnp.where(lane == sub * WAVE + jj, acc + jnp.sum(t), acc)
        out_v[pl.ds(g16 * L, L)] = acc
        return 0

    lax.fori_loop(0, NG16, group16, 0)

    pltpu.sync_copy(out_v, out_hbm.at[pl.ds(base, B_PER_W)])


def kernel(user, item, embed_user, embed_item, W, b):
    mesh = plsc.VectorSubcoreMesh(core_axis_name="c", subcore_axis_name="s")
    run = pl.kernel(
        _gmf_kernel,
        mesh=mesh,
        compiler_params=pltpu.CompilerParams(
            needs_layout_passes=False, use_tc_tiling_on_sc=True
        ),
        out_type=jax.ShapeDtypeStruct((B,), jnp.float32),
        scratch_types=[
            pltpu.VMEM((IDX_PAD,), jnp.int32),
            pltpu.VMEM((IDX_PAD,), jnp.int32),
            pltpu.VMEM((F,), jnp.int32),
            pltpu.VMEM((WAVE, F, 128), jnp.float32),
            pltpu.VMEM((WAVE, F, 128), jnp.float32),
            pltpu.VMEM((B_PER_W,), jnp.float32),
            pltpu.VMEM((1, F), jnp.float32),
            pltpu.VMEM((L,), jnp.float32),
            pltpu.SemaphoreType.DMA,
        ],
    )
    b16 = jnp.broadcast_to(b.astype(jnp.float32), (L,))
    return run(embed_user.T, embed_item.T, user, item, W, b16)


# speedup vs baseline: 20.5130x; 1.0866x over previous
import jax
import jax.numpy as jnp
from jax import lax
from jax.experimental import pallas as pl
from jax.experimental.pallas import tpu as pltpu
from jax.experimental.pallas import tpu_sc as plsc

B = 16384
F = 32
L = 16

_info = plsc.get_sparse_core_info()
NC, NS = _info.num_cores, _info.num_subcores
NW = NC * NS
B_PER_W = B // NW
WAVE = 8
NG16 = B_PER_W // L
IDX_PAD = B_PER_W + L


def _gmf_kernel(eu_hbm, ei_hbm, user_hbm, item_hbm, w_hbm, b_hbm, out_hbm,
                uidx_v, iidx_v, fidx_v, eu_sb, ei_sb, out_v, w_v, b_v, sem):
    wid = lax.axis_index("s") * NC + lax.axis_index("c")
    base = wid * B_PER_W

    pltpu.sync_copy(w_hbm, w_v)
    pltpu.sync_copy(b_hbm, b_v)
    pltpu.sync_copy(user_hbm.at[pl.ds(base, B_PER_W)],
                    uidx_v.at[pl.ds(0, B_PER_W)])
    pltpu.sync_copy(item_hbm.at[pl.ds(base, B_PER_W)],
                    iidx_v.at[pl.ds(0, B_PER_W)])

    lane = lax.iota(jnp.int32, L)
    fidx_v[pl.ds(0, L)] = lane
    fidx_v[pl.ds(L, L)] = lane + L

    w_lo = w_v[0, pl.ds(0, L)]
    w_hi = w_v[0, pl.ds(L, L)]
    bias_v = b_v[...]
    zeros = jnp.zeros((L,), jnp.int32)
    f_lo = lane
    f_hi = lane + L

    def group16(g16, _):
        acc = bias_v
        for sub in range(2):
            g = g16 * 2 + sub
            iu = uidx_v[pl.ds(g * WAVE, L)]
            ii = iidx_v[pl.ds(g * WAVE, L)]
            qu = (iu >> 7) * 128
            cu = iu & 127
            qi = (ii >> 7) * 128
            ci = ii & 127
            for jj in range(WAVE):
                pltpu.async_copy(
                    eu_hbm.at[:, pl.ds(pl.multiple_of(qu[jj], 128), 128)],
                    eu_sb.at[jj], sem,
                )
                pltpu.async_copy(
                    ei_hbm.at[:, pl.ds(pl.multiple_of(qi[jj], 128), 128)],
                    ei_sb.at[jj], sem,
                )
            for jj in range(WAVE):
                pltpu.make_async_copy(
                    eu_hbm.at[:, pl.ds(pl.multiple_of(qu[jj], 128), 128)],
                    eu_sb.at[jj], sem,
                ).wait()
                pltpu.make_async_copy(
                    ei_hbm.at[:, pl.ds(pl.multiple_of(qi[jj], 128), 128)],
                    ei_sb.at[jj], sem,
                ).wait()
            for jj in range(WAVE):
                jj_vec = jnp.full((L,), jj, jnp.int32)
                cu_vec = zeros + cu[jj]
                ci_vec = zeros + ci[jj]
                eu_l = plsc.load_gather(eu_sb, [jj_vec, f_lo, cu_vec])
                eu_h = plsc.load_gather(eu_sb, [jj_vec, f_hi, cu_vec])
                ei_l = plsc.load_gather(ei_sb, [jj_vec, f_lo, ci_vec])
                ei_h = plsc.load_gather(ei_sb, [jj_vec, f_hi, ci_vec])
                t = (eu_l * ei_l) * w_lo + (eu_h * ei_h) * w_hi
                acc = jnp.where(lane == sub * WAVE + jj, acc + jnp.sum(t), acc)
        out_v[pl.ds(g16 * L, L)] = acc
        return 0

    lax.fori_loop(0, NG16, group16, 0)

    pltpu.sync_copy(out_v, out_hbm.at[pl.ds(base, B_PER_W)])


def kernel(user, item, embed_user, embed_item, W, b):
    mesh = plsc.VectorSubcoreMesh(core_axis_name="c", subcore_axis_name="s")
    run = pl.kernel(
        _gmf_kernel,
        mesh=mesh,
        compiler_params=pltpu.CompilerParams(
            needs_layout_passes=False, use_tc_tiling_on_sc=True
        ),
        out_type=jax.ShapeDtypeStruct((B,), jnp.float32),
        scratch_types=[
            pltpu.VMEM((IDX_PAD,), jnp.int32),
            pltpu.VMEM((IDX_PAD,), jnp.int32),
            pltpu.VMEM((F,), jnp.int32),
            pltpu.VMEM((WAVE, F, 128), jnp.float32),
            pltpu.VMEM((WAVE, F, 128), jnp.float32),
            pltpu.VMEM((B_PER_W,), jnp.float32),
            pltpu.VMEM((1, F), jnp.float32),
            pltpu.VMEM((L,), jnp.float32),
            pltpu.SemaphoreType.DMA,
        ],
    )
    b16 = jnp.broadcast_to(b.astype(jnp.float32), (L,))
    return run(embed_user.T, embed_item.T, user, item, W, b16)


# speedup vs baseline: 23.5591x; 1.1485x over previous
import jax
import jax.numpy as jnp
from jax import lax
from jax.experimental import pallas as pl
from jax.experimental.pallas import tpu as pltpu
from jax.experimental.pallas import tpu_sc as plsc

B = 16384
F = 32
L = 16

_info = plsc.get_sparse_core_info()
NC, NS = _info.num_cores, _info.num_subcores
NW = NC * NS
B_PER_W = B // NW
WAVE = 4
NG16 = B_PER_W // L
IDX_PAD = B_PER_W + L


def _gmf_kernel(eu_hbm, ei_hbm, user_hbm, item_hbm, w_hbm, b_hbm, out_hbm,
                uidx_v, iidx_v, fidx_v, eu_sb, ei_sb, out_v, w_v, b_v, sem):
    wid = lax.axis_index("s") * NC + lax.axis_index("c")
    base = wid * B_PER_W

    pltpu.sync_copy(w_hbm, w_v)
    pltpu.sync_copy(b_hbm, b_v)
    pltpu.sync_copy(user_hbm.at[pl.ds(base, B_PER_W)],
                    uidx_v.at[pl.ds(0, B_PER_W)])
    pltpu.sync_copy(item_hbm.at[pl.ds(base, B_PER_W)],
                    iidx_v.at[pl.ds(0, B_PER_W)])

    lane = lax.iota(jnp.int32, L)
    fidx_v[pl.ds(0, L)] = lane
    fidx_v[pl.ds(L, L)] = lane + L

    w_lo = w_v[0, pl.ds(0, L)]
    w_hi = w_v[0, pl.ds(L, L)]
    bias_v = b_v[...]
    zeros = jnp.zeros((L,), jnp.int32)
    f_lo = lane
    f_hi = lane + L

    def fire(w, p, qu, qi):
        for jj in range(WAVE):
            pltpu.async_copy(
                eu_hbm.at[:, pl.ds(pl.multiple_of(qu[w * WAVE + jj], 128), 128)],
                eu_sb.at[p, jj], sem,
            )
            pltpu.async_copy(
                ei_hbm.at[:, pl.ds(pl.multiple_of(qi[w * WAVE + jj], 128), 128)],
                ei_sb.at[p, jj], sem,
            )

    def drain(w, p, qu, qi):
        for jj in range(WAVE):
            pltpu.make_async_copy(
                eu_hbm.at[:, pl.ds(pl.multiple_of(qu[w * WAVE + jj], 128), 128)],
                eu_sb.at[p, jj], sem,
            ).wait()
            pltpu.make_async_copy(
                ei_hbm.at[:, pl.ds(pl.multiple_of(qi[w * WAVE + jj], 128), 128)],
                ei_sb.at[p, jj], sem,
            ).wait()

    def process(w, p, cu, ci, acc):
        for jj in range(WAVE):
            p_vec = jnp.full((L,), p, jnp.int32)
            jj_vec = jnp.full((L,), jj, jnp.int32)
            cu_vec = zeros + cu[w * WAVE + jj]
            ci_vec = zeros + ci[w * WAVE + jj]
            eu_l = plsc.load_gather(eu_sb, [p_vec, jj_vec, f_lo, cu_vec])
            eu_h = plsc.load_gather(eu_sb, [p_vec, jj_vec, f_hi, cu_vec])
            ei_l = plsc.load_gather(ei_sb, [p_vec, jj_vec, f_lo, ci_vec])
            ei_h = plsc.load_gather(ei_sb, [p_vec, jj_vec, f_hi, ci_vec])
            t = (eu_l * ei_l) * w_lo + (eu_h * ei_h) * w_hi
            acc = jnp.where(lane == w * WAVE + jj, acc + jnp.sum(t), acc)
        return acc

    def group16(g16, _):
        iu = uidx_v[pl.ds(g16 * L, L)]
        ii = iidx_v[pl.ds(g16 * L, L)]
        qu = (iu >> 7) * 128
        cu = iu & 127
        qi = (ii >> 7) * 128
        ci = ii & 127
        acc = bias_v
        fire(0, 0, qu, qi)
        fire(1, 1, qu, qi)
        drain(0, 0, qu, qi)
        acc = process(0, 0, cu, ci, acc)
        fire(2, 0, qu, qi)
        drain(1, 1, qu, qi)
        acc = process(1, 1, cu, ci, acc)
        fire(3, 1, qu, qi)
        drain(2, 0, qu, qi)
        acc = process(2, 0, cu, ci, acc)
        drain(3, 1, qu, qi)
        acc = process(3, 1, cu, ci, acc)
        out_v[pl.ds(g16 * L, L)] = acc
        return 0

    lax.fori_loop(0, NG16, group16, 0)

    pltpu.sync_copy(out_v, out_hbm.at[pl.ds(base, B_PER_W)])


def kernel(user, item, embed_user, embed_item, W, b):
    mesh = plsc.VectorSubcoreMesh(core_axis_name="c", subcore_axis_name="s")
    run = pl.kernel(
        _gmf_kernel,
        mesh=mesh,
        compiler_params=pltpu.CompilerParams(
            needs_layout_passes=False, use_tc_tiling_on_sc=True
        ),
        out_type=jax.ShapeDtypeStruct((B,), jnp.float32),
        scratch_types=[
            pltpu.VMEM((IDX_PAD,), jnp.int32),
            pltpu.VMEM((IDX_PAD,), jnp.int32),
            pltpu.VMEM((F,), jnp.int32),
            pltpu.VMEM((2, WAVE, F, 128), jnp.float32),
            pltpu.VMEM((2, WAVE, F, 128), jnp.float32),
            pltpu.VMEM((B_PER_W,), jnp.float32),
            pltpu.VMEM((1, F), jnp.float32),
            pltpu.VMEM((L,), jnp.float32),
            pltpu.SemaphoreType.DMA,
        ],
    )
    b16 = jnp.broadcast_to(b.astype(jnp.float32), (L,))
    return run(embed_user.T, embed_item.T, user, item, W, b16)


# speedup vs baseline: 23.6096x; 1.0021x over previous
import jax
import jax.numpy as jnp
from jax import lax
from jax.experimental import pallas as pl
from jax.experimental.pallas import tpu as pltpu
from jax.experimental.pallas import tpu_sc as plsc

B = 16384
F = 32
L = 16

_info = plsc.get_sparse_core_info()
NC, NS = _info.num_cores, _info.num_subcores
NW = NC * NS
B_PER_W = B // NW
WAVE = 4
NG16 = B_PER_W // L
IDX_PAD = B_PER_W + L


def _gmf_kernel(eu_hbm, ei_hbm, user_hbm, item_hbm, w_hbm, b_hbm, out_hbm,
                uidx_v, iidx_v, eu_sb, ei_sb, out_v, w_v, b_v, sem):
    wid = lax.axis_index("s") * NC + lax.axis_index("c")
    base = wid * B_PER_W

    pltpu.sync_copy(w_hbm, w_v)
    pltpu.sync_copy(b_hbm, b_v)
    pltpu.sync_copy(user_hbm.at[pl.ds(base, B_PER_W)],
                    uidx_v.at[pl.ds(0, B_PER_W)])
    pltpu.sync_copy(item_hbm.at[pl.ds(base, B_PER_W)],
                    iidx_v.at[pl.ds(0, B_PER_W)])

    lane = lax.iota(jnp.int32, L)

    w_lo = w_v[0, pl.ds(0, L)]
    w_hi = w_v[0, pl.ds(L, L)]
    bias_v = b_v[...]
    zeros = jnp.zeros((L,), jnp.int32)
    f_lo = lane
    f_hi = lane + L

    def fire(w, p, qu, qi):
        for jj in range(WAVE):
            pltpu.async_copy(
                eu_hbm.at[:, pl.ds(pl.multiple_of(qu[w * WAVE + jj], 128), 128)],
                eu_sb.at[p, jj], sem,
            )
            pltpu.async_copy(
                ei_hbm.at[:, pl.ds(pl.multiple_of(qi[w * WAVE + jj], 128), 128)],
                ei_sb.at[p, jj], sem,
            )

    def drain(w, p, qu, qi):
        for jj in range(WAVE):
            pltpu.make_async_copy(
                eu_hbm.at[:, pl.ds(pl.multiple_of(qu[w * WAVE + jj], 128), 128)],
                eu_sb.at[p, jj], sem,
            ).wait()
            pltpu.make_async_copy(
                ei_hbm.at[:, pl.ds(pl.multiple_of(qi[w * WAVE + jj], 128), 128)],
                ei_sb.at[p, jj], sem,
            ).wait()

    def process(w, p, cu, ci, acc):
        for jj in range(WAVE):
            p_vec = jnp.full((L,), p, jnp.int32)
            jj_vec = jnp.full((L,), jj, jnp.int32)
            cu_vec = zeros + cu[w * WAVE + jj]
            ci_vec = zeros + ci[w * WAVE + jj]
            eu_l = plsc.load_gather(eu_sb, [p_vec, jj_vec, f_lo, cu_vec])
            eu_h = plsc.load_gather(eu_sb, [p_vec, jj_vec, f_hi, cu_vec])
            ei_l = plsc.load_gather(ei_sb, [p_vec, jj_vec, f_lo, ci_vec])
            ei_h = plsc.load_gather(ei_sb, [p_vec, jj_vec, f_hi, ci_vec])
            t = (eu_l * ei_l) * w_lo + (eu_h * ei_h) * w_hi
            acc = jnp.where(lane == w * WAVE + jj, acc + jnp.sum(t), acc)
        return acc

    def group16(g16, _):
        iu = uidx_v[pl.ds(g16 * L, L)]
        ii = iidx_v[pl.ds(g16 * L, L)]
        qu = (iu >> 7) * 128
        cu = iu & 127
        qi = (ii >> 7) * 128
        ci = ii & 127
        acc = bias_v
        fire(0, 0, qu, qi)
        fire(1, 1, qu, qi)
        fire(2, 2, qu, qi)
        drain(0, 0, qu, qi)
        acc = process(0, 0, cu, ci, acc)
        fire(3, 0, qu, qi)
        drain(1, 1, qu, qi)
        acc = process(1, 1, cu, ci, acc)
        drain(2, 2, qu, qi)
        acc = process(2, 2, cu, ci, acc)
        drain(3, 0, qu, qi)
        acc = process(3, 0, cu, ci, acc)
        out_v[pl.ds(g16 * L, L)] = acc
        return 0

    lax.fori_loop(0, NG16, group16, 0)

    pltpu.sync_copy(out_v, out_hbm.at[pl.ds(base, B_PER_W)])


def kernel(user, item, embed_user, embed_item, W, b):
    mesh = plsc.VectorSubcoreMesh(core_axis_name="c", subcore_axis_name="s")
    run = pl.kernel(
        _gmf_kernel,
        mesh=mesh,
        compiler_params=pltpu.CompilerParams(
            needs_layout_passes=False, use_tc_tiling_on_sc=True
        ),
        out_type=jax.ShapeDtypeStruct((B,), jnp.float32),
        scratch_types=[
            pltpu.VMEM((IDX_PAD,), jnp.int32),
            pltpu.VMEM((IDX_PAD,), jnp.int32),
            pltpu.VMEM((3, WAVE, F, 128), jnp.float32),
            pltpu.VMEM((3, WAVE, F, 128), jnp.float32),
            pltpu.VMEM((B_PER_W,), jnp.float32),
            pltpu.VMEM((1, F), jnp.float32),
            pltpu.VMEM((L,), jnp.float32),
            pltpu.SemaphoreType.DMA,
        ],
    )
    b16 = jnp.broadcast_to(b.astype(jnp.float32), (L,))
    return run(embed_user.T, embed_item.T, user, item, W, b16)
